# baseline (speedup 1.0000x reference)
import math

import jax
import jax.numpy as jnp
from jax.experimental import pallas as pl
from jax.experimental.pallas import tpu as pltpu

D_MODEL = 768
N_EXPERTS = 64
EXPERT_SIZE = 48
SEQ = 2048
SIZE = N_EXPERTS * EXPERT_SIZE
TB = 512
NT = SEQ // TB


def _moe_body(x_ref, es_ref, keys_ref, v2_ref, out_ref, reg_ref, acc_ref,
              k2_ref):
    i = pl.program_id(0)
    xb = x_ref[...]

    @pl.when(i == 0)
    def _():
        for g in range(N_EXPERTS // 8):
            chunk = jnp.concatenate(
                [keys_ref[8 * g + j] for j in range(8)], axis=1)
            k2_ref[:, pl.ds(g * 384, 384)] = chunk

    sel_raw = jax.lax.dot_general(
        xb, es_ref[...], (((1,), (1,)), ((), ())),
        preferred_element_type=jnp.float32)

    row_max = jnp.max(sel_raw, axis=1, keepdims=True)
    lse = row_max + jnp.log(jnp.sum(jnp.exp(sel_raw - row_max), axis=1,
                                    keepdims=True))
    p = jnp.exp(sel_raw - lse)
    colsum = jnp.sum(p, axis=0, keepdims=True)

    @pl.when(i == 0)
    def _():
        acc_ref[...] = jnp.zeros_like(acc_ref)

    acc_ref[...] += colsum

    sel = jax.nn.sigmoid(sel_raw)
    eidx = jax.lax.broadcasted_iota(jnp.int32, (TB, N_EXPERTS), 1)
    m1 = jnp.max(sel, axis=1, keepdims=True)
    i1 = jnp.min(jnp.where(sel == m1, eidx, N_EXPERTS), axis=1, keepdims=True)
    sel2 = jnp.where(eidx == i1, -jnp.inf, sel)
    m2 = jnp.max(sel2, axis=1, keepdims=True)
    i2 = jnp.min(jnp.where(sel2 == m2, eidx, N_EXPERTS), axis=1, keepdims=True)

    s = jax.lax.dot_general(
        xb, k2_ref[...], (((1,), (0,)), ((), ())),
        preferred_element_type=jnp.float32)
    cexp = jax.lax.broadcasted_iota(jnp.int32, (TB, SIZE), 1) // EXPERT_SIZE
    w = (jnp.where(cexp == i1, m1, 0.0) + jnp.where(cexp == i2, m2, 0.0))

    s = jnp.maximum(s, 0.0) * w
    out_ref[...] = jax.lax.dot_general(
        s, v2_ref[...], (((1,), (0,)), ((), ())),
        preferred_element_type=jnp.float32)

    @pl.when(i == NT - 1)
    def _():
        acc = acc_ref[...]
        lm = jnp.log(acc) - math.log(SEQ)
        contrib = jnp.where(acc > 0.0, lm * (acc / SEQ), 0.0)
        reg_ref[...] = jnp.sum(contrib).reshape(1, 1)


def kernel(x, keys, values, expert_sel):
    xs = x.reshape(SEQ, D_MODEL)
    v2 = values.reshape(SIZE, D_MODEL)
    res, reg = pl.pallas_call(
        _moe_body,
        grid=(NT,),
        in_specs=[
            pl.BlockSpec((TB, D_MODEL), lambda i: (i, 0)),
            pl.BlockSpec((N_EXPERTS, D_MODEL), lambda i: (0, 0)),
            pl.BlockSpec((N_EXPERTS, D_MODEL, EXPERT_SIZE),
                         lambda i: (0, 0, 0)),
            pl.BlockSpec((SIZE, D_MODEL), lambda i: (0, 0)),
        ],
        out_specs=[
            pl.BlockSpec((TB, D_MODEL), lambda i: (i, 0)),
            pl.BlockSpec((1, 1), lambda i: (0, 0)),
        ],
        out_shape=[
            jax.ShapeDtypeStruct((SEQ, D_MODEL), jnp.float32),
            jax.ShapeDtypeStruct((1, 1), jnp.float32),
        ],
        scratch_shapes=[pltpu.VMEM((1, N_EXPERTS), jnp.float32),
                        pltpu.VMEM((D_MODEL, SIZE), jnp.float32)],
    )(xs, expert_sel, keys, v2)
    return res.reshape(x.shape), reg.reshape(())


# speedup vs baseline: 1.4484x; 1.4484x over previous
import math

import jax
import jax.numpy as jnp
from jax.experimental import pallas as pl
from jax.experimental.pallas import tpu as pltpu

D_MODEL = 768
N_EXPERTS = 64
EXPERT_SIZE = 48
SEQ = 2048
SIZE = N_EXPERTS * EXPERT_SIZE
TB = 1024
NT = SEQ // TB


def _moe_body(x_ref, es_ref, k2_ref, v2_ref, out_ref, reg_ref, acc_ref):
    i = pl.program_id(0)
    xb = x_ref[...]

    sel_raw = jax.lax.dot_general(
        xb, es_ref[...], (((1,), (1,)), ((), ())),
        preferred_element_type=jnp.float32)

    row_max = jnp.max(sel_raw, axis=1, keepdims=True)
    lse = row_max + jnp.log(jnp.sum(jnp.exp(sel_raw - row_max), axis=1,
                                    keepdims=True))
    p = jnp.exp(sel_raw - lse)
    colsum = jnp.sum(p, axis=0, keepdims=True)

    @pl.when(i == 0)
    def _():
        acc_ref[...] = jnp.zeros_like(acc_ref)

    acc_ref[...] += colsum

    sel = jax.nn.sigmoid(sel_raw)
    eidx = jax.lax.broadcasted_iota(jnp.int32, (TB, N_EXPERTS), 1)
    m1 = jnp.max(sel, axis=1, keepdims=True)
    i1 = jnp.min(jnp.where(sel == m1, eidx, N_EXPERTS), axis=1, keepdims=True)
    sel2 = jnp.where(eidx == i1, -jnp.inf, sel)
    m2 = jnp.max(sel2, axis=1, keepdims=True)
    i2 = jnp.min(jnp.where(sel2 == m2, eidx, N_EXPERTS), axis=1, keepdims=True)

    s = jax.lax.dot_general(
        xb, k2_ref[...], (((1,), (0,)), ((), ())),
        preferred_element_type=jnp.float32)
    cexp = jax.lax.broadcasted_iota(jnp.int32, (TB, SIZE), 1) // EXPERT_SIZE
    w = jnp.where(cexp == i1, m1, jnp.where(cexp == i2, m2, 0.0))

    s = jnp.maximum(s, 0.0) * w
    out_ref[...] = jax.lax.dot_general(
        s, v2_ref[...], (((1,), (0,)), ((), ())),
        preferred_element_type=jnp.float32)

    @pl.when(i == NT - 1)
    def _():
        acc = acc_ref[...]
        lm = jnp.log(acc) - math.log(SEQ)
        contrib = jnp.where(acc > 0.0, lm * (acc / SEQ), 0.0)
        reg_ref[...] = jnp.sum(contrib).reshape(1, 1)


def kernel(x, keys, values, expert_sel):
    xs = x.reshape(SEQ, D_MODEL)
    k2 = jnp.transpose(keys, (1, 0, 2)).reshape(D_MODEL, SIZE)
    v2 = values.reshape(SIZE, D_MODEL)
    res, reg = pl.pallas_call(
        _moe_body,
        grid=(NT,),
        in_specs=[
            pl.BlockSpec((TB, D_MODEL), lambda i: (i, 0)),
            pl.BlockSpec((N_EXPERTS, D_MODEL), lambda i: (0, 0)),
            pl.BlockSpec((D_MODEL, SIZE), lambda i: (0, 0)),
            pl.BlockSpec((SIZE, D_MODEL), lambda i: (0, 0)),
        ],
        out_specs=[
            pl.BlockSpec((TB, D_MODEL), lambda i: (i, 0)),
            pl.BlockSpec((1, 1), lambda i: (0, 0)),
        ],
        out_shape=[
            jax.ShapeDtypeStruct((SEQ, D_MODEL), jnp.float32),
            jax.ShapeDtypeStruct((1, 1), jnp.float32),
        ],
        scratch_shapes=[pltpu.VMEM((1, N_EXPERTS), jnp.float32)],
    )(xs, expert_sel, k2, v2)
    return res.reshape(x.shape), reg.reshape(())
